# baseline (speedup 1.0000x reference)
import functools

import jax
import jax.numpy as jnp
from jax import lax
from jax.experimental import pallas as pl
from jax.experimental.pallas import tpu as pltpu
from jax.experimental.pallas import tpu_sc as plsc

_F = 26
_V = 100000
_D = 32
_B = 16384

_NW = 32
_WIN = _B // _NW
_HALF = _WIN // 2
_QROWS = 4

_mesh = plsc.VectorSubcoreMesh(core_axis_name="c", subcore_axis_name="s")


@functools.partial(
    pl.kernel,
    mesh=_mesh,
    compiler_params=pltpu.CompilerParams(needs_layout_passes=False),
    out_type=jax.ShapeDtypeStruct((_F, _D, _B), jnp.float32),
    scratch_types=[
        pltpu.VMEM((2, _QROWS, 128), jnp.int32),
        pltpu.VMEM((2, _WIN), jnp.int32),
        pltpu.VMEM((2, _HALF, 128), jnp.float32),
        pltpu.VMEM((4, 4, 8, 128), jnp.float32),
        pltpu.SemaphoreType.DMA,
        pltpu.SemaphoreType.DMA,
        pltpu.SemaphoreType.DMA,
        pltpu.SemaphoreType.DMA,
    ],
)
def _embed_kernel(q_hbm, s_hbm, table_hbm, out_hbm,
                  qbuf, sbuf, gbuf, tbuf, gsem0, gsem1, osem, xsem):
    wid = lax.axis_index("s") * 2 + lax.axis_index("c")
    b0 = wid * _WIN
    gsem = (gsem0, gsem1)
    iota16 = lax.iota(jnp.int32, 16)

    def stage_idx(f, issue):
        p = f & 1
        mk = pltpu.async_copy if issue else pltpu.make_async_copy
        descs = [mk(q_hbm.at[f, pl.ds(b0 + j * 128, 128)], qbuf.at[p, j], xsem)
                 for j in range(_QROWS)]
        descs.append(mk(s_hbm.at[f, pl.ds(b0, _WIN)], sbuf.at[p], xsem))
        return descs

    def gathers(f, h, issue):
        p = f & 1
        mk = pltpu.async_copy if issue else pltpu.make_async_copy
        return [mk(table_hbm.at[qbuf.at[p, h * 2 + j]],
                   gbuf.at[h, pl.ds(j * 128, 128)], gsem[h])
                for j in range(2)]

    def out_writes(f, issue):
        mk = pltpu.async_copy if issue else pltpu.make_async_copy
        return [mk(tbuf.at[dg, tb],
                   out_hbm.at[f, pl.ds(dg * 8, 8), pl.ds(b0 + tb * 128, 128)],
                   osem)
                for dg in range(4) for tb in range(4)]

    def extract_half(f, h):
        p = f & 1
        gh = gbuf.at[h]

        def pack_body(m, _):
            tb = h * 2 + m // 8
            kk = m - (m // 8) * 8
            row = m * 16 + iota16
            sub = sbuf[p, pl.ds(h * _HALF + m * 16, 16)]
            for dg in range(4):
                for dd in range(8):
                    col = sub + (dg * 8 + dd)
                    val = plsc.load_gather(gh, [row, col])
                    tbuf[dg, tb, dd, pl.ds(kk * 16, 16)] = val
            return 0

        lax.fori_loop(0, 16, pack_body, 0)

    for d in stage_idx(0, True):
        d.wait()
    gathers(0, 0, True)
    gathers(0, 1, True)
    stage_idx(1, True)

    def field_body(f, _):
        @pl.when(f + 1 < _F)
        def _():
            for d in stage_idx(f + 1, False):
                d.wait()
        for d in gathers(f, 0, False):
            d.wait()

        @pl.when(f >= 1)
        def _():
            for d in out_writes(f - 1, False):
                d.wait()
        extract_half(f, 0)
        for d in gathers(f, 1, False):
            d.wait()
        extract_half(f, 1)

        @pl.when(f + 1 < _F)
        def _():
            gathers(f + 1, 0, True)
            gathers(f + 1, 1, True)

        @pl.when(f + 2 < _F)
        def _():
            stage_idx(f + 2, True)
        out_writes(f, True)
        return 0

    lax.fori_loop(0, _F, field_body, 0)
    for d in out_writes(_F - 1, False):
        d.wait()


def kernel(X, tables):
    xt = jnp.transpose(X).astype(jnp.int32)
    offs = (jnp.arange(_F, dtype=jnp.int32) * (_V // 4))[:, None]
    q = (xt >> 2) + offs
    s = (xt & 3) * _D
    table4 = tables.reshape(_F * _V // 4, 128)
    out = _embed_kernel(q, s, table4)
    return jnp.transpose(out, (2, 0, 1))


# speedup vs baseline: 1.0478x; 1.0478x over previous
import functools

import jax
import jax.numpy as jnp
from jax import lax
from jax.experimental import pallas as pl
from jax.experimental.pallas import tpu as pltpu
from jax.experimental.pallas import tpu_sc as plsc

_F = 26
_V = 100000
_D = 32
_B = 16384

_NW = 32
_WIN = _B // _NW
_NQ = 4

_mesh = plsc.VectorSubcoreMesh(core_axis_name="c", subcore_axis_name="s")


@functools.partial(
    pl.kernel,
    mesh=_mesh,
    compiler_params=pltpu.CompilerParams(needs_layout_passes=False),
    out_type=jax.ShapeDtypeStruct((_F, _D, _B), jnp.float32),
    scratch_types=[
        pltpu.VMEM((2, _NQ, 128), jnp.int32),
        pltpu.VMEM((2, _WIN), jnp.int32),
        pltpu.VMEM((2, 128, 128), jnp.float32),
        pltpu.VMEM((_NQ, _NQ, 8, 128), jnp.float32),
        pltpu.SemaphoreType.DMA,
        pltpu.SemaphoreType.DMA,
        pltpu.SemaphoreType.DMA,
        pltpu.SemaphoreType.DMA,
    ],
)
def _embed_kernel(q_hbm, s_hbm, table_hbm, out_hbm,
                  qbuf, sbuf, gbuf, tbuf, gsem0, gsem1, osem, xsem):
    wid = lax.axis_index("s") * 2 + lax.axis_index("c")
    b0 = wid * _WIN
    gsem = (gsem0, gsem1)
    iota16 = lax.iota(jnp.int32, 16)

    def stage_idx(f, issue):
        p = f & 1
        mk = pltpu.async_copy if issue else pltpu.make_async_copy
        descs = [mk(q_hbm.at[f, pl.ds(b0 + j * 128, 128)], qbuf.at[p, j], xsem)
                 for j in range(_NQ)]
        descs.append(mk(s_hbm.at[f, pl.ds(b0, _WIN)], sbuf.at[p], xsem))
        return descs

    def gather(f, qt, issue):
        p = f & 1
        mk = pltpu.async_copy if issue else pltpu.make_async_copy
        return mk(table_hbm.at[qbuf.at[p, qt]], gbuf.at[qt & 1],
                  gsem[qt & 1])

    def out_writes(f, issue):
        mk = pltpu.async_copy if issue else pltpu.make_async_copy
        return [mk(tbuf.at[dg, tb],
                   out_hbm.at[f, pl.ds(dg * 8, 8), pl.ds(b0 + tb * 128, 128)],
                   osem)
                for dg in range(4) for tb in range(4)]

    def extract_quarter(f, qt):
        p = f & 1
        eb = gbuf.at[qt & 1]

        def pack_body(m, _):
            row = m * 16 + iota16
            sub = sbuf[p, pl.ds(qt * 128 + m * 16, 16)]
            for dg in range(4):
                for dd in range(8):
                    col = sub + (dg * 8 + dd)
                    val = plsc.load_gather(eb, [row, col])
                    tbuf[dg, qt, dd, pl.ds(m * 16, 16)] = val
            return 0

        lax.fori_loop(0, 8, pack_body, 0)

    for d in stage_idx(0, True):
        d.wait()
    gather(0, 0, True)
    gather(0, 1, True)
    stage_idx(1, True)

    def field_body(f, _):
        @pl.when(f + 1 < _F)
        def _():
            for d in stage_idx(f + 1, False):
                d.wait()

        for qt in range(_NQ):
            gather(f, qt, False).wait()
            if qt == 0:
                @pl.when(f >= 1)
                def _():
                    for d in out_writes(f - 1, False):
                        d.wait()
            extract_quarter(f, qt)
            if qt + 2 < _NQ:
                gather(f, qt + 2, True)
            else:
                @pl.when(f + 1 < _F)
                def _(qt=qt):
                    gather(f + 1, qt - 2, True)

        @pl.when(f + 2 < _F)
        def _():
            stage_idx(f + 2, True)
        out_writes(f, True)
        return 0

    lax.fori_loop(0, _F, field_body, 0)
    for d in out_writes(_F - 1, False):
        d.wait()


def kernel(X, tables):
    xt = jnp.transpose(X).astype(jnp.int32)
    offs = (jnp.arange(_F, dtype=jnp.int32) * (_V // 4))[:, None]
    q = (xt >> 2) + offs
    s = (xt & 3) * _D
    table4 = tables.reshape(_F * _V // 4, 128)
    out = _embed_kernel(q, s, table4)
    return jnp.transpose(out, (2, 0, 1))
